# baseline (speedup 1.0000x reference)
import dataclasses
import functools

import jax
import jax.numpy as jnp
from jax import lax
from jax.experimental import pallas as pl
from jax.experimental.pallas import tpu as pltpu
from jax.experimental.pallas import tpu_sc as plsc

NUM_CORES = 2
NUM_SUBCORES = 16
NUM_WORKERS = NUM_CORES * NUM_SUBCORES
LANES = 16
EMB = 64
GATHER_CHUNK = 128


def _sc_fused(user, item, user_emb, item_emb, wvec):
    B = user.shape[0]
    bpw = B // NUM_WORKERS
    n_chunks = bpw // GATHER_CHUNK
    mesh = plsc.VectorSubcoreMesh(core_axis_name="c", subcore_axis_name="s")
    cp = pltpu.CompilerParams()
    for fld, val in (("needs_layout_passes", False),
                     ("use_tc_tiling_on_sc", False)):
        if fld in pltpu.CompilerParams.__dataclass_fields__:
            cp = dataclasses.replace(cp, **{fld: val})

    @functools.partial(
        pl.kernel,
        out_type=jax.ShapeDtypeStruct((B,), jnp.float32),
        mesh=mesh,
        compiler_params=cp,
        scratch_types=[
            pltpu.VMEM((bpw,), jnp.int32),
            pltpu.VMEM((bpw,), jnp.int32),
            pltpu.VMEM((bpw, EMB), jnp.float32),
            pltpu.VMEM((bpw, EMB), jnp.float32),
            pltpu.VMEM((144,), jnp.float32),
            pltpu.VMEM((bpw,), jnp.float32),
            pltpu.SemaphoreType.DMA,
            pltpu.SemaphoreType.DMA,
        ],
    )
    def k(user_h, item_h, uemb_h, iemb_h, w_h, out_h,
          uidx_v, iidx_v, urows_v, irows_v, w_v, score_v, sem_u, sem_i):
        wid = lax.axis_index("s") * NUM_CORES + lax.axis_index("c")
        base = wid * bpw
        pltpu.sync_copy(w_h, w_v)
        pltpu.sync_copy(user_h.at[pl.ds(base, bpw)], uidx_v)
        pltpu.sync_copy(item_h.at[pl.ds(base, bpw)], iidx_v)
        copies = []
        for c in range(n_chunks):
            sl = pl.ds(c * GATHER_CHUNK, GATHER_CHUNK)
            copies.append(pltpu.async_copy(
                uemb_h.at[uidx_v.at[sl]], urows_v.at[sl], sem_u))
            copies.append(pltpu.async_copy(
                iemb_h.at[iidx_v.at[sl]], irows_v.at[sl], sem_i))
        for cp in copies:
            cp.wait()

        bias = w_v[pl.ds(2 * EMB, LANES)]
        wchunks = [w_v[pl.ds(LANES * j, LANES)] for j in range(2 * EMB // LANES)]
        lane_iota = lax.iota(jnp.int32, LANES)

        @pl.loop(0, bpw // LANES)
        def _(g):
            row_idx = lane_iota + g * LANES
            acc = bias
            for d in range(EMB):
                col = plsc.load_gather(
                    urows_v, [row_idx, jnp.full((LANES,), d, jnp.int32)])
                acc = acc + col * wchunks[d // LANES][d % LANES]
            for d in range(EMB):
                col = plsc.load_gather(
                    irows_v, [row_idx, jnp.full((LANES,), d, jnp.int32)])
                acc = acc + col * wchunks[4 + d // LANES][d % LANES]
            score_v[pl.ds(g * LANES, LANES)] = 1.0 / (1.0 + jnp.exp(-acc))

        pltpu.sync_copy(score_v, out_h.at[pl.ds(base, bpw)])

    return k(user, item, user_emb, item_emb, wvec)


def kernel(user, item, user_emb, item_emb, fc_w, fc_b):
    w = fc_w.reshape(-1).astype(jnp.float32)
    wvec = jnp.concatenate(
        [w, jnp.broadcast_to(fc_b.astype(jnp.float32), (LANES,))])
    out = _sc_fused(user.astype(jnp.int32), item.astype(jnp.int32),
                    user_emb, item_emb, wvec)
    return out.reshape(-1, 1)


# speedup vs baseline: 1.0067x; 1.0067x over previous
import dataclasses
import functools

import jax
import jax.numpy as jnp
from jax import lax
from jax.experimental import pallas as pl
from jax.experimental.pallas import tpu as pltpu
from jax.experimental.pallas import tpu_sc as plsc

NUM_CORES = 2
NUM_SUBCORES = 16
NUM_WORKERS = NUM_CORES * NUM_SUBCORES
LANES = 16
EMB = 64
ROW2 = 2 * EMB
CHUNK = 128


def _sc_fused(user, item, uemb2, iemb2, wvec):
    B = user.shape[0]
    bpw = B // NUM_WORKERS
    n_chunks = bpw // CHUNK
    mesh = plsc.VectorSubcoreMesh(core_axis_name="c", subcore_axis_name="s")
    cp = pltpu.CompilerParams()
    if "needs_layout_passes" in pltpu.CompilerParams.__dataclass_fields__:
        cp = dataclasses.replace(cp, needs_layout_passes=False)

    @functools.partial(
        pl.kernel,
        out_type=jax.ShapeDtypeStruct((B,), jnp.float32),
        mesh=mesh,
        compiler_params=cp,
        scratch_types=[
            pltpu.VMEM((bpw,), jnp.int32),
            pltpu.VMEM((bpw,), jnp.int32),
            pltpu.VMEM((bpw,), jnp.int32),
            pltpu.VMEM((bpw,), jnp.int32),
            pltpu.VMEM((CHUNK, ROW2), jnp.float32),
            pltpu.VMEM((CHUNK, ROW2), jnp.float32),
            pltpu.VMEM((144,), jnp.float32),
            pltpu.VMEM((bpw,), jnp.float32),
            pltpu.SemaphoreType.DMA,
            pltpu.SemaphoreType.DMA,
        ],
    )
    def k(user_h, item_h, uemb_h, iemb_h, w_h, out_h,
          uidx_v, iidx_v, uidx2_v, iidx2_v, urows_v, irows_v, w_v, score_v,
          sem_u, sem_i):
        wid = lax.axis_index("s") * NUM_CORES + lax.axis_index("c")
        base = wid * bpw
        pltpu.sync_copy(w_h, w_v)
        pltpu.sync_copy(user_h.at[pl.ds(base, bpw)], uidx_v)
        pltpu.sync_copy(item_h.at[pl.ds(base, bpw)], iidx_v)

        @pl.loop(0, bpw, step=LANES)
        def _(i):
            sl = pl.ds(i, LANES)
            uidx2_v[sl] = lax.shift_right_logical(uidx_v[sl], 1)
            iidx2_v[sl] = lax.shift_right_logical(iidx_v[sl], 1)

        bias = w_v[pl.ds(2 * EMB, LANES)]
        wchunks = [w_v[pl.ds(LANES * j, LANES)] for j in range(2 * EMB // LANES)]
        lane_iota = lax.iota(jnp.int32, LANES)

        @pl.loop(0, n_chunks)
        def _(c):
            csl = pl.ds(c * CHUNK, CHUNK)
            cp_u = pltpu.async_copy(uemb_h.at[uidx2_v.at[csl]], urows_v, sem_u)
            cp_i = pltpu.async_copy(iemb_h.at[iidx2_v.at[csl]], irows_v, sem_i)
            cp_u.wait()
            cp_i.wait()

            @pl.loop(0, CHUNK // LANES)
            def _(g):
                r0 = c * CHUNK + g * LANES
                rsl = pl.ds(r0, LANES)
                ucol = (uidx_v[rsl] & 1) * EMB
                icol = (iidx_v[rsl] & 1) * EMB
                row_idx = lane_iota + g * LANES
                acc = bias
                for d in range(EMB):
                    cu = plsc.load_gather(urows_v, [row_idx, ucol + d])
                    acc = acc + cu * wchunks[d // LANES][d % LANES]
                    ci = plsc.load_gather(irows_v, [row_idx, icol + d])
                    acc = acc + ci * wchunks[4 + d // LANES][d % LANES]
                score_v[rsl] = 1.0 / (1.0 + jnp.exp(-acc))

        pltpu.sync_copy(score_v, out_h.at[pl.ds(base, bpw)])

    return k(user, item, uemb2, iemb2, wvec)


def kernel(user, item, user_emb, item_emb, fc_w, fc_b):
    w = fc_w.reshape(-1).astype(jnp.float32)
    wvec = jnp.concatenate(
        [w, jnp.broadcast_to(fc_b.astype(jnp.float32), (LANES,))])
    uemb2 = user_emb.reshape(-1, ROW2)
    iemb2 = item_emb.reshape(-1, ROW2)
    out = _sc_fused(user.astype(jnp.int32), item.astype(jnp.int32),
                    uemb2, iemb2, wvec)
    return out.reshape(-1, 1)


# speedup vs baseline: 4.1590x; 4.1312x over previous
import dataclasses
import functools

import jax
import jax.numpy as jnp
from jax import lax
from jax.experimental import pallas as pl
from jax.experimental.pallas import tpu as pltpu
from jax.experimental.pallas import tpu_sc as plsc

NUM_CORES = 2
NUM_SUBCORES = 16
NUM_WORKERS = NUM_CORES * NUM_SUBCORES
LANES = 16
EMB = 64
BC = 4096
GCHUNK = 128


def _tc_scan_scores(uT, iT, wu, wi):
    V = uT.shape[1]
    grid = (pl.cdiv(V, BC),)

    def body(uT_ref, iT_ref, wu_ref, wi_ref, us_ref, is_ref):
        us_ref[...] = jnp.sum(uT_ref[...] * wu_ref[...], axis=0)
        is_ref[...] = jnp.sum(iT_ref[...] * wi_ref[...], axis=0)

    return pl.pallas_call(
        body,
        grid=grid,
        in_specs=[
            pl.BlockSpec((EMB, BC), lambda j: (0, j)),
            pl.BlockSpec((EMB, BC), lambda j: (0, j)),
            pl.BlockSpec((EMB, 1), lambda j: (0, 0)),
            pl.BlockSpec((EMB, 1), lambda j: (0, 0)),
        ],
        out_specs=[
            pl.BlockSpec((BC,), lambda j: (j,)),
            pl.BlockSpec((BC,), lambda j: (j,)),
        ],
        out_shape=[
            jax.ShapeDtypeStruct((V,), jnp.float32),
            jax.ShapeDtypeStruct((V,), jnp.float32),
        ],
    )(uT, iT, wu, wi)


def _sc_gather_sigmoid(user, item, u_score, i_score, bias16):
    B = user.shape[0]
    bpw = B // NUM_WORKERS
    n_chunks = bpw // GCHUNK
    mesh = plsc.VectorSubcoreMesh(core_axis_name="c", subcore_axis_name="s")
    cp = pltpu.CompilerParams()
    if "needs_layout_passes" in pltpu.CompilerParams.__dataclass_fields__:
        cp = dataclasses.replace(cp, needs_layout_passes=False)

    @functools.partial(
        pl.kernel,
        out_type=jax.ShapeDtypeStruct((B,), jnp.float32),
        mesh=mesh,
        compiler_params=cp,
        scratch_types=[
            pltpu.VMEM((bpw,), jnp.int32),
            pltpu.VMEM((bpw,), jnp.int32),
            pltpu.VMEM((bpw,), jnp.float32),
            pltpu.VMEM((bpw,), jnp.float32),
            pltpu.VMEM((LANES,), jnp.float32),
            pltpu.SemaphoreType.DMA,
            pltpu.SemaphoreType.DMA,
        ],
    )
    def k(user_h, item_h, us_h, is_h, b_h, out_h,
          uidx_v, iidx_v, uval_v, ival_v, b_v, sem_u, sem_i):
        wid = lax.axis_index("s") * NUM_CORES + lax.axis_index("c")
        base = wid * bpw
        pltpu.sync_copy(b_h, b_v)
        pltpu.sync_copy(user_h.at[pl.ds(base, bpw)], uidx_v)
        pltpu.sync_copy(item_h.at[pl.ds(base, bpw)], iidx_v)
        copies = []
        for c in range(n_chunks):
            sl = pl.ds(c * GCHUNK, GCHUNK)
            copies.append(pltpu.async_copy(
                us_h.at[uidx_v.at[sl]], uval_v.at[sl], sem_u))
            copies.append(pltpu.async_copy(
                is_h.at[iidx_v.at[sl]], ival_v.at[sl], sem_i))
        for cpy in copies:
            cpy.wait()
        bias = b_v[pl.ds(0, LANES)]

        @pl.loop(0, bpw, step=LANES)
        def _(i):
            sl = pl.ds(i, LANES)
            x = uval_v[sl] + ival_v[sl] + bias
            uval_v[sl] = 1.0 / (1.0 + jnp.exp(-x))

        pltpu.sync_copy(uval_v, out_h.at[pl.ds(base, bpw)])

    return k(user, item, u_score, i_score, bias16)


def kernel(user, item, user_emb, item_emb, fc_w, fc_b):
    w = fc_w.reshape(-1).astype(jnp.float32)
    wu = w[:EMB].reshape(EMB, 1)
    wi = w[EMB:].reshape(EMB, 1)
    u_score, i_score = _tc_scan_scores(user_emb.T, item_emb.T, wu, wi)
    bias16 = jnp.broadcast_to(fc_b.astype(jnp.float32), (LANES,))
    out = _sc_gather_sigmoid(user.astype(jnp.int32), item.astype(jnp.int32),
                             u_score, i_score, bias16)
    return out.reshape(-1, 1)


# speedup vs baseline: 6.3881x; 1.5360x over previous
import dataclasses
import functools

import jax
import jax.numpy as jnp
from jax import lax
from jax.experimental import pallas as pl
from jax.experimental.pallas import tpu as pltpu
from jax.experimental.pallas import tpu_sc as plsc

NUM_CORES = 2
NUM_SUBCORES = 16
NUM_WORKERS = NUM_CORES * NUM_SUBCORES
LANES = 16
EMB = 64
BC = 16384
GCHUNK = 128


def _tc_scan_scores(uT, iT, wu, wi):
    V = uT.shape[1]
    grid = (pl.cdiv(V, BC),)

    def body(uT_ref, iT_ref, wu_ref, wi_ref, us_ref, is_ref):
        us_ref[...] = jnp.sum(uT_ref[...] * wu_ref[...], axis=0)
        is_ref[...] = jnp.sum(iT_ref[...] * wi_ref[...], axis=0)

    return pl.pallas_call(
        body,
        grid=grid,
        in_specs=[
            pl.BlockSpec((EMB, BC), lambda j: (0, j)),
            pl.BlockSpec((EMB, BC), lambda j: (0, j)),
            pl.BlockSpec((EMB, 1), lambda j: (0, 0)),
            pl.BlockSpec((EMB, 1), lambda j: (0, 0)),
        ],
        out_specs=[
            pl.BlockSpec((BC,), lambda j: (j,)),
            pl.BlockSpec((BC,), lambda j: (j,)),
        ],
        out_shape=[
            jax.ShapeDtypeStruct((V,), jnp.float32),
            jax.ShapeDtypeStruct((V,), jnp.float32),
        ],
    )(uT, iT, wu, wi)


def _sc_gather_sigmoid(user, item, u_score, i_score, bias16):
    B = user.shape[0]
    bpw = B // NUM_WORKERS
    n_chunks = bpw // GCHUNK
    mesh = plsc.VectorSubcoreMesh(core_axis_name="c", subcore_axis_name="s")
    cp = pltpu.CompilerParams()
    if "needs_layout_passes" in pltpu.CompilerParams.__dataclass_fields__:
        cp = dataclasses.replace(cp, needs_layout_passes=False)

    @functools.partial(
        pl.kernel,
        out_type=jax.ShapeDtypeStruct((B,), jnp.float32),
        mesh=mesh,
        compiler_params=cp,
        scratch_types=[
            pltpu.VMEM((bpw,), jnp.int32),
            pltpu.VMEM((bpw,), jnp.int32),
            pltpu.VMEM((bpw,), jnp.float32),
            pltpu.VMEM((bpw,), jnp.float32),
            pltpu.VMEM((LANES,), jnp.float32),
            pltpu.SemaphoreType.DMA,
            pltpu.SemaphoreType.DMA,
        ],
    )
    def k(user_h, item_h, us_h, is_h, b_h, out_h,
          uidx_v, iidx_v, uval_v, ival_v, b_v, sem_u, sem_i):
        wid = lax.axis_index("s") * NUM_CORES + lax.axis_index("c")
        base = wid * bpw
        pltpu.sync_copy(b_h, b_v)
        pltpu.sync_copy(user_h.at[pl.ds(base, bpw)], uidx_v)
        pltpu.sync_copy(item_h.at[pl.ds(base, bpw)], iidx_v)
        copies = []
        for c in range(n_chunks):
            sl = pl.ds(c * GCHUNK, GCHUNK)
            copies.append(pltpu.async_copy(
                us_h.at[uidx_v.at[sl]], uval_v.at[sl], sem_u))
            copies.append(pltpu.async_copy(
                is_h.at[iidx_v.at[sl]], ival_v.at[sl], sem_i))
        for cpy in copies:
            cpy.wait()
        bias = b_v[pl.ds(0, LANES)]

        @pl.loop(0, bpw, step=LANES)
        def _(i):
            sl = pl.ds(i, LANES)
            x = uval_v[sl] + ival_v[sl] + bias
            uval_v[sl] = 1.0 / (1.0 + jnp.exp(-x))

        pltpu.sync_copy(uval_v, out_h.at[pl.ds(base, bpw)])

    return k(user, item, u_score, i_score, bias16)


def kernel(user, item, user_emb, item_emb, fc_w, fc_b):
    w = fc_w.reshape(-1).astype(jnp.float32)
    wu = w[:EMB].reshape(EMB, 1)
    wi = w[EMB:].reshape(EMB, 1)
    u_score, i_score = _tc_scan_scores(user_emb.T, item_emb.T, wu, wi)
    bias16 = jnp.broadcast_to(fc_b.astype(jnp.float32), (LANES,))
    out = _sc_gather_sigmoid(user.astype(jnp.int32), item.astype(jnp.int32),
                             u_score, i_score, bias16)
    return out.reshape(-1, 1)


# speedup vs baseline: 6.4175x; 1.0046x over previous
import dataclasses
import functools

import jax
import jax.numpy as jnp
from jax import lax
from jax.experimental import pallas as pl
from jax.experimental.pallas import tpu as pltpu
from jax.experimental.pallas import tpu_sc as plsc

NUM_CORES = 2
NUM_SUBCORES = 16
NUM_WORKERS = NUM_CORES * NUM_SUBCORES
LANES = 16
EMB = 64
BC = 32768
GCHUNK = 128


def _tc_scan_scores(uT, iT, wu, wi):
    V = uT.shape[1]
    grid = (pl.cdiv(V, BC),)

    def body(uT_ref, iT_ref, wu_ref, wi_ref, us_ref, is_ref):
        us_ref[...] = jnp.sum(uT_ref[...] * wu_ref[...], axis=0)
        is_ref[...] = jnp.sum(iT_ref[...] * wi_ref[...], axis=0)

    return pl.pallas_call(
        body,
        grid=grid,
        in_specs=[
            pl.BlockSpec((EMB, BC), lambda j: (0, j)),
            pl.BlockSpec((EMB, BC), lambda j: (0, j)),
            pl.BlockSpec((EMB, 1), lambda j: (0, 0)),
            pl.BlockSpec((EMB, 1), lambda j: (0, 0)),
        ],
        out_specs=[
            pl.BlockSpec((BC,), lambda j: (j,)),
            pl.BlockSpec((BC,), lambda j: (j,)),
        ],
        out_shape=[
            jax.ShapeDtypeStruct((V,), jnp.float32),
            jax.ShapeDtypeStruct((V,), jnp.float32),
        ],
    )(uT, iT, wu, wi)


def _sc_gather_sigmoid(user, item, u_score, i_score, bias16):
    B = user.shape[0]
    bpw = B // NUM_WORKERS
    n_chunks = bpw // GCHUNK
    mesh = plsc.VectorSubcoreMesh(core_axis_name="c", subcore_axis_name="s")
    cp = pltpu.CompilerParams()
    if "needs_layout_passes" in pltpu.CompilerParams.__dataclass_fields__:
        cp = dataclasses.replace(cp, needs_layout_passes=False)

    @functools.partial(
        pl.kernel,
        out_type=jax.ShapeDtypeStruct((B,), jnp.float32),
        mesh=mesh,
        compiler_params=cp,
        scratch_types=[
            pltpu.VMEM((bpw,), jnp.int32),
            pltpu.VMEM((bpw,), jnp.int32),
            pltpu.VMEM((bpw,), jnp.float32),
            pltpu.VMEM((bpw,), jnp.float32),
            pltpu.VMEM((LANES,), jnp.float32),
            pltpu.SemaphoreType.DMA,
            pltpu.SemaphoreType.DMA,
        ],
    )
    def k(user_h, item_h, us_h, is_h, b_h, out_h,
          uidx_v, iidx_v, uval_v, ival_v, b_v, sem_u, sem_i):
        wid = lax.axis_index("s") * NUM_CORES + lax.axis_index("c")
        base = wid * bpw
        pltpu.sync_copy(b_h, b_v)
        pltpu.sync_copy(user_h.at[pl.ds(base, bpw)], uidx_v)
        pltpu.sync_copy(item_h.at[pl.ds(base, bpw)], iidx_v)
        copies = []
        for c in range(n_chunks):
            sl = pl.ds(c * GCHUNK, GCHUNK)
            copies.append(pltpu.async_copy(
                us_h.at[uidx_v.at[sl]], uval_v.at[sl], sem_u))
            copies.append(pltpu.async_copy(
                is_h.at[iidx_v.at[sl]], ival_v.at[sl], sem_i))
        for cpy in copies:
            cpy.wait()
        bias = b_v[pl.ds(0, LANES)]

        @pl.loop(0, bpw, step=LANES)
        def _(i):
            sl = pl.ds(i, LANES)
            x = uval_v[sl] + ival_v[sl] + bias
            uval_v[sl] = 1.0 / (1.0 + jnp.exp(-x))

        pltpu.sync_copy(uval_v, out_h.at[pl.ds(base, bpw)])

    return k(user, item, u_score, i_score, bias16)


def kernel(user, item, user_emb, item_emb, fc_w, fc_b):
    w = fc_w.reshape(-1).astype(jnp.float32)
    wu = w[:EMB].reshape(EMB, 1)
    wi = w[EMB:].reshape(EMB, 1)
    u_score, i_score = _tc_scan_scores(user_emb.T, item_emb.T, wu, wi)
    bias16 = jnp.broadcast_to(fc_b.astype(jnp.float32), (LANES,))
    out = _sc_gather_sigmoid(user.astype(jnp.int32), item.astype(jnp.int32),
                             u_score, i_score, bias16)
    return out.reshape(-1, 1)
